# Initial kernel scaffold; baseline (speedup 1.0000x reference)
import jax
import jax.numpy as jnp
from jax.experimental import pallas as pl


def kernel(x, edge_index, batch, params):
    raise NotImplementedError("write your pallas kernel here")


# speedup vs baseline: 4.4224x; 4.4224x over previous
import functools

import jax
import jax.numpy as jnp
from jax import lax
from jax.experimental import pallas as pl
from jax.experimental.pallas import tpu as pltpu
from jax.experimental.pallas import tpu_sc as plsc

N = 10000
E = 320000
D = 128
H = 128
L = 4
G = 64

NC = 2
NS = 16
NW = NC * NS
CH = 80

NPAD = 10240
NB = 10
BN_ROWS = N // NB


def _make_agg_kernel():
    e_per_w = E // NW
    n_ch = e_per_w // CH
    rows_per_tile = NPAD // NS
    mesh = plsc.VectorSubcoreMesh(
        core_axis_name="c", subcore_axis_name="s", num_cores=NC, num_subcores=NS
    )

    @functools.partial(
        pl.kernel,
        mesh=mesh,
        out_type=jax.ShapeDtypeStruct((NC, NPAD, D), jnp.float32),
        scratch_types=[
            pltpu.VMEM((CH,), jnp.int32),
            pltpu.VMEM((CH,), jnp.int32),
            pltpu.VMEM((CH, D), jnp.float32),
            pltpu.VMEM_SHARED((NPAD, D), jnp.float32),
            pltpu.SemaphoreType.DMA,
        ],
    )
    def agg(h_hbm, src_hbm, dst_hbm, zeros_hbm, out_hbm,
            src_v, dst_v, rows_v, acc_sh, sem):
        cid = lax.axis_index("c")
        sid = lax.axis_index("s")
        wid = sid * NC + cid
        r0 = sid * rows_per_tile

        @pl.when(cid == 0)
        def _():
            pltpu.sync_copy(h_hbm.at[pl.ds(r0, rows_per_tile)],
                            acc_sh.at[pl.ds(r0, rows_per_tile)])

        @pl.when(cid != 0)
        def _():
            pltpu.sync_copy(zeros_hbm.at[pl.ds(r0, rows_per_tile)],
                            acc_sh.at[pl.ds(r0, rows_per_tile)])

        plsc.subcore_barrier()

        ebase = wid * e_per_w

        def body(i, carry):
            off = ebase + i * CH
            pltpu.sync_copy(src_hbm.at[pl.ds(off, CH)], src_v)
            pltpu.sync_copy(dst_hbm.at[pl.ds(off, CH)], dst_v)
            pltpu.async_copy(h_hbm.at[src_v], rows_v, sem).wait()
            pltpu.sync_copy(rows_v, acc_sh.at[dst_v], add=True)
            return carry

        lax.fori_loop(0, n_ch, body, 0)
        plsc.subcore_barrier()

        pltpu.sync_copy(acc_sh.at[pl.ds(r0, rows_per_tile)],
                        out_hbm.at[cid].at[pl.ds(r0, rows_per_tile)])

    return agg


_AGG_CACHE = {}


def _agg_call(h, src, dst, zeros):
    if "k" not in _AGG_CACHE:
        _AGG_CACHE["k"] = _make_agg_kernel()
    return _AGG_CACHE["k"](h, src, dst, zeros)


def _in_proj(x, W, b):
    def body(x_ref, w_ref, b_ref, o_ref):
        o_ref[...] = jnp.maximum(
            jnp.dot(x_ref[...], w_ref[...], preferred_element_type=jnp.float32)
            + b_ref[...], 0.0)

    return pl.pallas_call(
        body,
        grid=(NB,),
        in_specs=[
            pl.BlockSpec((BN_ROWS, D), lambda i: (i, 0)),
            pl.BlockSpec((D, H), lambda i: (0, 0)),
            pl.BlockSpec((1, H), lambda i: (0, 0)),
        ],
        out_specs=pl.BlockSpec((BN_ROWS, H), lambda i: (i, 0)),
        out_shape=jax.ShapeDtypeStruct((NPAD, H), jnp.float32),
    )(x, W, b.reshape(1, H))


def _mlp_stats(parts, W1, b1, W2, b2):
    def body(p0_ref, p1_ref, w1_ref, b1_ref, w2_ref, b2_ref,
             z_ref, s_ref, q_ref):
        i = pl.program_id(0)
        u = p0_ref[0] + p1_ref[0]
        t = jnp.maximum(
            jnp.dot(u, w1_ref[...], preferred_element_type=jnp.float32)
            + b1_ref[...], 0.0)
        z = jnp.dot(t, w2_ref[...], preferred_element_type=jnp.float32) + b2_ref[...]
        z_ref[...] = z

        @pl.when(i == 0)
        def _():
            s_ref[...] = jnp.zeros_like(s_ref)
            q_ref[...] = jnp.zeros_like(q_ref)

        s_ref[...] += jnp.sum(z, axis=0, keepdims=True)
        q_ref[...] += jnp.sum(z * z, axis=0, keepdims=True)

    return pl.pallas_call(
        body,
        grid=(NB,),
        in_specs=[
            pl.BlockSpec((1, BN_ROWS, H), lambda i: (0, i, 0)),
            pl.BlockSpec((1, BN_ROWS, H), lambda i: (1, i, 0)),
            pl.BlockSpec((H, 2 * H), lambda i: (0, 0)),
            pl.BlockSpec((1, 2 * H), lambda i: (0, 0)),
            pl.BlockSpec((2 * H, H), lambda i: (0, 0)),
            pl.BlockSpec((1, H), lambda i: (0, 0)),
        ],
        out_specs=[
            pl.BlockSpec((BN_ROWS, H), lambda i: (i, 0)),
            pl.BlockSpec((1, H), lambda i: (0, 0)),
            pl.BlockSpec((1, H), lambda i: (0, 0)),
        ],
        out_shape=[
            jax.ShapeDtypeStruct((N, H), jnp.float32),
            jax.ShapeDtypeStruct((1, H), jnp.float32),
            jax.ShapeDtypeStruct((1, H), jnp.float32),
        ],
    )(parts, parts, W1, b1.reshape(1, 2 * H), W2, b2.reshape(1, H))


def _bn_relu_pool(z, s, q, bn_w, bn_b, onehot):
    def body(z_ref, s_ref, q_ref, w_ref, bb_ref, p_ref, h_ref, pool_ref):
        i = pl.program_id(0)
        mean = s_ref[...] * (1.0 / N)
        var = q_ref[...] * (1.0 / N) - mean * mean
        inv = lax.rsqrt(var + 1e-5)
        scale = w_ref[...] * inv
        shift = bb_ref[...] - mean * scale
        hn = jnp.maximum(z_ref[...] * scale + shift, 0.0)
        h_ref[...] = hn

        @pl.when(i == 0)
        def _():
            pool_ref[...] = jnp.zeros_like(pool_ref)

        pool_ref[...] += lax.dot_general(
            p_ref[...], hn, (((0,), (0,)), ((), ())),
            preferred_element_type=jnp.float32)

    return pl.pallas_call(
        body,
        grid=(NB,),
        in_specs=[
            pl.BlockSpec((BN_ROWS, H), lambda i: (i, 0)),
            pl.BlockSpec((1, H), lambda i: (0, 0)),
            pl.BlockSpec((1, H), lambda i: (0, 0)),
            pl.BlockSpec((1, H), lambda i: (0, 0)),
            pl.BlockSpec((1, H), lambda i: (0, 0)),
            pl.BlockSpec((BN_ROWS, G), lambda i: (i, 0)),
        ],
        out_specs=[
            pl.BlockSpec((BN_ROWS, H), lambda i: (i, 0)),
            pl.BlockSpec((G, H), lambda i: (0, 0)),
        ],
        out_shape=[
            jax.ShapeDtypeStruct((NPAD, H), jnp.float32),
            jax.ShapeDtypeStruct((G, H), jnp.float32),
        ],
    )(z, s, q, bn_w.reshape(1, H), bn_b.reshape(1, H), onehot)


def _readout(g, Ws1, bs1, Ws2, bs2, Wh1cat, bh1cat, Wh2blk, bh2row):
    NH = Wh2blk.shape[1]

    def body(g_ref, ws1_ref, bs1_ref, ws2_ref, bs2_ref,
             wh1_ref, bh1_ref, wh2_ref, bh2_ref, o_ref):
        s = jnp.maximum(
            jnp.dot(g_ref[...], ws1_ref[...], preferred_element_type=jnp.float32)
            + bs1_ref[...], 0.0)
        s = jnp.maximum(
            jnp.dot(s, ws2_ref[...], preferred_element_type=jnp.float32)
            + bs2_ref[...], 0.0)
        t = jnp.maximum(
            jnp.dot(s, wh1_ref[...], preferred_element_type=jnp.float32)
            + bh1_ref[...], 0.0)
        o_ref[...] = (jnp.dot(t, wh2_ref[...], preferred_element_type=jnp.float32)
                      + bh2_ref[...])

    return pl.pallas_call(
        body,
        out_shape=jax.ShapeDtypeStruct((G, NH), jnp.float32),
    )(g, Ws1, bs1.reshape(1, -1), Ws2, bs2.reshape(1, -1),
      Wh1cat, bh1cat.reshape(1, -1), Wh2blk, bh2row.reshape(1, -1))


def kernel(x, edge_index, batch, params):
    src = edge_index[0]
    dst = edge_index[1]
    zeros = jnp.zeros((NPAD, D), jnp.float32)
    onehot = (batch[:, None] == jnp.arange(G, dtype=batch.dtype)[None, :]
              ).astype(jnp.float32)

    h = _in_proj(x, params['in_W'], params['in_b'])

    pooled = []
    for l in range(L):
        p = params['layers'][l]
        parts = _agg_call(h, src, dst, zeros)
        z, s, q = _mlp_stats(parts, p['W1'], p['b1'], p['W2'], p['b2'])
        h, pool_l = _bn_relu_pool(z, s, q, p['bn_w'], p['bn_b'], onehot)
        pooled.append(pool_l)

    g = jnp.concatenate(pooled, axis=1)

    Wh1cat = jnp.concatenate([hd['W1'] for hd in params['heads']], axis=1)
    bh1cat = jnp.concatenate([hd['b1'] for hd in params['heads']])
    Wh2blk = jax.scipy.linalg.block_diag(*[hd['W2'] for hd in params['heads']])
    bh2row = jnp.concatenate([hd['b2'] for hd in params['heads']])

    out = _readout(g, params['Ws1'], params['bs1'], params['Ws2'], params['bs2'],
                   Wh1cat, bh1cat, Wh2blk, bh2row)
    return out.T
